# baseline (speedup 1.0000x reference)
import functools

import jax
import jax.numpy as jnp
from jax import lax
from jax.experimental import pallas as pl
from jax.experimental.pallas import tpu as pltpu
from jax.experimental.pallas import tpu_sc as plsc

_NUM_CORES = 2
_NUM_SUBCORES = 16
_NW = _NUM_CORES * _NUM_SUBCORES


def _tc_pack_sum(mt, qt, half, block_k):
    D, V = mt.shape
    grid = half // block_k
    off = half // block_k

    def body(ma_ref, qa_ref, mb_ref, qb_ref, o_ref):
        eye = (lax.broadcasted_iota(jnp.int32, (D, D), 0)
               == lax.broadcasted_iota(jnp.int32, (D, D), 1)).astype(jnp.float32)
        dn = (((0,), (0,)), ((), ()))
        ta = lax.dot_general(ma_ref[...] + qa_ref[...], eye, dn,
                             preferred_element_type=jnp.float32)
        tb = lax.dot_general(mb_ref[...] + qb_ref[...], eye, dn,
                             preferred_element_type=jnp.float32)
        o_ref[...] = jnp.concatenate([ta, tb], axis=1)

    return pl.pallas_call(
        body,
        grid=(grid,),
        in_specs=[
            pl.BlockSpec((D, block_k), lambda i: (0, i)),
            pl.BlockSpec((D, block_k), lambda i: (0, i)),
            pl.BlockSpec((D, block_k), lambda i: (0, i + off)),
            pl.BlockSpec((D, block_k), lambda i: (0, i + off)),
        ],
        out_specs=pl.BlockSpec((block_k, 2 * D), lambda i: (i, 0)),
        out_shape=jax.ShapeDtypeStruct((half, 2 * D), jnp.float32),
    )(mt, qt, mt, qt)


def _sc_gather_packed(sum2, idx, half):
    B = idx.shape[0]
    L2 = sum2.shape[1]
    b_per_w = B // _NW
    assert B % (8 * _NW) == 0

    mesh = plsc.VectorSubcoreMesh(core_axis_name="c", subcore_axis_name="s")

    @functools.partial(
        pl.kernel,
        out_type=jax.ShapeDtypeStruct((B, L2), jnp.float32),
        mesh=mesh,
        scratch_types=[
            pltpu.VMEM((b_per_w,), jnp.int32),
            pltpu.VMEM((b_per_w,), jnp.int32),
            pltpu.VMEM((b_per_w, L2), jnp.float32),
            pltpu.SemaphoreType.DMA,
        ],
    )
    def gather_kernel(sum2_hbm, idx_hbm, g2_hbm, idx_v, idx2_v, g_v, sem):
        wid = lax.axis_index("s") * _NUM_CORES + lax.axis_index("c")
        base = wid * b_per_w
        pltpu.sync_copy(idx_hbm.at[pl.ds(base, b_per_w)], idx_v)
        for i in range(b_per_w // 16):
            v = idx_v[pl.ds(i * 16, 16)]
            idx2_v[pl.ds(i * 16, 16)] = jnp.where(v >= half, v - half, v)
        pltpu.async_copy(sum2_hbm.at[idx2_v], g_v, sem).wait()
        pltpu.sync_copy(g_v, g2_hbm.at[pl.ds(base, b_per_w)])

    return gather_kernel(sum2, idx)


def _tc_subtract_t(pt, g2, idx, half, n_chunks, depth):
    T, D, B = pt.shape
    ch = T // n_chunks

    def body(p_hbm, g2_ref, idx_ref, o_hbm, pbuf, obuf, in_sems, out_sems):
        par = idx_ref[...][:, None]
        sel = jnp.where(par >= half, g2_ref[:, D:2 * D], g2_ref[:, 0:D])
        c = jnp.transpose(sel * 0.5, (1, 0))

        in_cps = [
            pltpu.make_async_copy(
                p_hbm.at[pl.ds(j * ch, ch)], pbuf.at[j % depth],
                in_sems.at[j % depth])
            for j in range(n_chunks)
        ]
        out_cps = [
            pltpu.make_async_copy(
                obuf.at[j % depth], o_hbm.at[pl.ds(j * ch, ch)],
                out_sems.at[j % depth])
            for j in range(n_chunks)
        ]
        for j in range(depth):
            in_cps[j].start()
        for j in range(n_chunks):
            in_cps[j].wait()
            if j >= depth:
                out_cps[j - depth].wait()
            obuf[j % depth] = pbuf[j % depth] - c[None, :, :]
            out_cps[j].start()
            if j + depth < n_chunks:
                in_cps[j + depth].start()
        for j in range(n_chunks - depth, n_chunks):
            out_cps[j].wait()

    return pl.pallas_call(
        body,
        in_specs=[
            pl.BlockSpec(memory_space=pl.ANY),
            pl.BlockSpec((B, 2 * D), lambda: (0, 0)),
            pl.BlockSpec((B,), lambda: (0,)),
        ],
        out_specs=pl.BlockSpec(memory_space=pl.ANY),
        out_shape=jax.ShapeDtypeStruct((T, D, B), jnp.float32),
        scratch_shapes=[
            pltpu.VMEM((depth, ch, D, B), jnp.float32),
            pltpu.VMEM((depth, ch, D, B), jnp.float32),
            pltpu.SemaphoreType.DMA((depth,)),
            pltpu.SemaphoreType.DMA((depth,)),
        ],
    )(pt, g2, idx)


def kernel(prosody, spkr_id, means, question):
    idx = spkr_id.astype(jnp.int32)
    mt = jnp.transpose(means, (1, 0))
    qt = jnp.transpose(question, (1, 0))
    half = 51200
    sum2 = _tc_pack_sum(mt, qt, half=half, block_k=3200)
    g2 = _sc_gather_packed(sum2, idx, half=half)
    pt = jnp.transpose(prosody, (1, 2, 0))
    out_t = _tc_subtract_t(pt, g2, idx, half=half, n_chunks=25, depth=6)
    return jnp.transpose(out_t, (2, 0, 1))


# speedup vs baseline: 1.1075x; 1.1075x over previous
import functools

import jax
import jax.numpy as jnp
from jax import lax
from jax.experimental import pallas as pl
from jax.experimental.pallas import tpu as pltpu
from jax.experimental.pallas import tpu_sc as plsc

_NUM_CORES = 2
_NUM_SUBCORES = 16
_NW = _NUM_CORES * _NUM_SUBCORES


def _tc_pack_sum(mt, qt, half, block_k):
    D, V = mt.shape
    grid = half // block_k
    off = half // block_k

    del off
    depth = 3
    v_edge = (V // 128) * 128

    def body(mt_hbm, qt_hbm, o_hbm, mabuf, qabuf, mbbuf, qbbuf, obuf,
             in_sems, out_sems):
        def in_slice(tab, col0, buf, slot, sem):
            width = min(block_k, v_edge - col0) if col0 < v_edge else 0
            if width <= 0:
                return None
            return pltpu.make_async_copy(
                tab.at[:, pl.ds(col0, width)],
                buf.at[slot, :, pl.ds(0, width)], sem)

        in_cps = []
        for j in range(grid):
            slot = j % depth
            cps = [
                in_slice(mt_hbm, j * block_k, mabuf, slot, in_sems.at[slot, 0]),
                in_slice(qt_hbm, j * block_k, qabuf, slot, in_sems.at[slot, 1]),
                in_slice(mt_hbm, half + j * block_k, mbbuf, slot, in_sems.at[slot, 2]),
                in_slice(qt_hbm, half + j * block_k, qbbuf, slot, in_sems.at[slot, 3]),
            ]
            in_cps.append([c for c in cps if c is not None])
        out_cps = [
            pltpu.make_async_copy(
                obuf.at[j % depth], o_hbm.at[pl.ds(j * block_k, block_k)],
                out_sems.at[j % depth])
            for j in range(grid)
        ]
        for j in range(depth):
            for c in in_cps[j]:
                c.start()
        for j in range(grid):
            slot = j % depth
            for c in in_cps[j]:
                c.wait()
            if j >= depth:
                out_cps[j - depth].wait()
            ta = jnp.transpose(mabuf[slot] + qabuf[slot], (1, 0))
            tb = jnp.transpose(mbbuf[slot] + qbbuf[slot], (1, 0))
            obuf[slot] = jnp.concatenate([ta, tb], axis=1)
            out_cps[j].start()
            if j + depth < grid:
                for c in in_cps[j + depth]:
                    c.start()
        for j in range(grid - depth, grid):
            out_cps[j].wait()

    return pl.pallas_call(
        body,
        in_specs=[
            pl.BlockSpec(memory_space=pl.ANY),
            pl.BlockSpec(memory_space=pl.ANY),
        ],
        out_specs=pl.BlockSpec(memory_space=pl.ANY),
        out_shape=jax.ShapeDtypeStruct((half, 2 * D), jnp.float32),
        scratch_shapes=[
            pltpu.VMEM((depth, D, block_k), jnp.float32),
            pltpu.VMEM((depth, D, block_k), jnp.float32),
            pltpu.VMEM((depth, D, block_k), jnp.float32),
            pltpu.VMEM((depth, D, block_k), jnp.float32),
            pltpu.VMEM((depth, block_k, 2 * D), jnp.float32),
            pltpu.SemaphoreType.DMA((depth, 4)),
            pltpu.SemaphoreType.DMA((depth,)),
        ],
    )(mt, qt)


def _sc_gather_packed(sum2, idx, half):
    B = idx.shape[0]
    L2 = sum2.shape[1]
    b_per_w = B // _NW
    assert B % (8 * _NW) == 0

    mesh = plsc.VectorSubcoreMesh(core_axis_name="c", subcore_axis_name="s")

    @functools.partial(
        pl.kernel,
        out_type=jax.ShapeDtypeStruct((B, L2), jnp.float32),
        mesh=mesh,
        scratch_types=[
            pltpu.VMEM((b_per_w,), jnp.int32),
            pltpu.VMEM((b_per_w,), jnp.int32),
            pltpu.VMEM((b_per_w, L2), jnp.float32),
            pltpu.SemaphoreType.DMA,
        ],
    )
    def gather_kernel(sum2_hbm, idx_hbm, g2_hbm, idx_v, idx2_v, g_v, sem):
        wid = lax.axis_index("s") * _NUM_CORES + lax.axis_index("c")
        base = wid * b_per_w
        pltpu.sync_copy(idx_hbm.at[pl.ds(base, b_per_w)], idx_v)
        for i in range(b_per_w // 16):
            v = idx_v[pl.ds(i * 16, 16)]
            idx2_v[pl.ds(i * 16, 16)] = jnp.where(v >= half, v - half, v)
        pltpu.async_copy(sum2_hbm.at[idx2_v], g_v, sem).wait()
        pltpu.sync_copy(g_v, g2_hbm.at[pl.ds(base, b_per_w)])

    return gather_kernel(sum2, idx)


def _tc_subtract_t(pt, g2, idx, tail_mq, half, v_edge, n_chunks, depth):
    T, D, B = pt.shape
    n_tail = tail_mq.shape[1]
    ch = T // n_chunks

    def body(p_hbm, g2_ref, idx_ref, tail_ref, o_hbm, pbuf, obuf,
             in_sems, out_sems):
        par = idx_ref[...][:, None]
        sel = jnp.where(par >= half, g2_ref[:, D:2 * D], g2_ref[:, 0:D])
        c = jnp.transpose(sel * 0.5, (1, 0))
        oh = (lax.broadcasted_iota(jnp.int32, (n_tail, B), 0)
              == (idx_ref[...] - v_edge)[None, :]).astype(jnp.float32)
        cfix = lax.dot_general(tail_ref[...] * 0.5, oh,
                               (((1,), (0,)), ((), ())),
                               preferred_element_type=jnp.float32)
        is_tail = (idx_ref[...] >= v_edge)[None, :]
        c = jnp.where(is_tail, cfix, c)

        in_cps = [
            pltpu.make_async_copy(
                p_hbm.at[pl.ds(j * ch, ch)], pbuf.at[j % depth],
                in_sems.at[j % depth])
            for j in range(n_chunks)
        ]
        out_cps = [
            pltpu.make_async_copy(
                obuf.at[j % depth], o_hbm.at[pl.ds(j * ch, ch)],
                out_sems.at[j % depth])
            for j in range(n_chunks)
        ]
        for j in range(depth):
            in_cps[j].start()
        for j in range(n_chunks):
            in_cps[j].wait()
            if j >= depth:
                out_cps[j - depth].wait()
            obuf[j % depth] = pbuf[j % depth] - c[None, :, :]
            out_cps[j].start()
            if j + depth < n_chunks:
                in_cps[j + depth].start()
        for j in range(n_chunks - depth, n_chunks):
            out_cps[j].wait()

    return pl.pallas_call(
        body,
        in_specs=[
            pl.BlockSpec(memory_space=pl.ANY),
            pl.BlockSpec((B, 2 * D), lambda: (0, 0)),
            pl.BlockSpec((B,), lambda: (0,)),
            pl.BlockSpec((D, n_tail), lambda: (0, 0)),
        ],
        out_specs=pl.BlockSpec(memory_space=pl.ANY),
        out_shape=jax.ShapeDtypeStruct((T, D, B), jnp.float32),
        scratch_shapes=[
            pltpu.VMEM((depth, ch, D, B), jnp.float32),
            pltpu.VMEM((depth, ch, D, B), jnp.float32),
            pltpu.SemaphoreType.DMA((depth,)),
            pltpu.SemaphoreType.DMA((depth,)),
        ],
    )(pt, g2, idx, tail_mq)


def kernel(prosody, spkr_id, means, question):
    idx = spkr_id.astype(jnp.int32)
    mt = jnp.transpose(means, (1, 0))
    qt = jnp.transpose(question, (1, 0))
    V = mt.shape[1]
    half = 51200
    v_edge = (V // 128) * 128
    sum2 = _tc_pack_sum(mt, qt, half=half, block_k=3200)
    g2 = _sc_gather_packed(sum2, idx, half=half)
    pt = jnp.transpose(prosody, (1, 2, 0))
    tail_mq = (lax.slice(mt, (0, v_edge), mt.shape)
               + lax.slice(qt, (0, v_edge), qt.shape))
    out_t = _tc_subtract_t(pt, g2, idx, tail_mq, half=half, v_edge=v_edge,
                           n_chunks=25, depth=6)
    return jnp.transpose(out_t, (2, 0, 1))


# speedup vs baseline: 1.1083x; 1.0007x over previous
import functools

import jax
import jax.numpy as jnp
from jax import lax
from jax.experimental import pallas as pl
from jax.experimental.pallas import tpu as pltpu
from jax.experimental.pallas import tpu_sc as plsc

_NUM_CORES = 2
_NUM_SUBCORES = 16
_NW = _NUM_CORES * _NUM_SUBCORES


def _tc_pack_sum(mt, qt, half, block_k):
    D, V = mt.shape
    grid = half // block_k
    off = half // block_k

    del off
    depth = 4
    v_edge = (V // 128) * 128

    def body(mt_hbm, qt_hbm, o_hbm, mabuf, qabuf, mbbuf, qbbuf, obuf,
             in_sems, out_sems):
        def in_slice(tab, col0, buf, slot, sem):
            width = min(block_k, v_edge - col0) if col0 < v_edge else 0
            if width <= 0:
                return None
            return pltpu.make_async_copy(
                tab.at[:, pl.ds(col0, width)],
                buf.at[slot, :, pl.ds(0, width)], sem)

        in_cps = []
        for j in range(grid):
            slot = j % depth
            cps = [
                in_slice(mt_hbm, j * block_k, mabuf, slot, in_sems.at[slot, 0]),
                in_slice(qt_hbm, j * block_k, qabuf, slot, in_sems.at[slot, 1]),
                in_slice(mt_hbm, half + j * block_k, mbbuf, slot, in_sems.at[slot, 2]),
                in_slice(qt_hbm, half + j * block_k, qbbuf, slot, in_sems.at[slot, 3]),
            ]
            in_cps.append([c for c in cps if c is not None])
        out_cps = [
            pltpu.make_async_copy(
                obuf.at[j % depth], o_hbm.at[pl.ds(j * block_k, block_k)],
                out_sems.at[j % depth])
            for j in range(grid)
        ]
        for j in range(depth):
            for c in in_cps[j]:
                c.start()
        for j in range(grid):
            slot = j % depth
            for c in in_cps[j]:
                c.wait()
            if j >= depth:
                out_cps[j - depth].wait()
            ta = jnp.transpose(mabuf[slot] + qabuf[slot], (1, 0))
            tb = jnp.transpose(mbbuf[slot] + qbbuf[slot], (1, 0))
            obuf[slot] = jnp.concatenate([ta, tb], axis=1)
            out_cps[j].start()
            if j + depth < grid:
                for c in in_cps[j + depth]:
                    c.start()
        for j in range(grid - depth, grid):
            out_cps[j].wait()

    return pl.pallas_call(
        body,
        in_specs=[
            pl.BlockSpec(memory_space=pl.ANY),
            pl.BlockSpec(memory_space=pl.ANY),
        ],
        out_specs=pl.BlockSpec(memory_space=pl.ANY),
        out_shape=jax.ShapeDtypeStruct((half, 2 * D), jnp.float32),
        scratch_shapes=[
            pltpu.VMEM((depth, D, block_k), jnp.float32),
            pltpu.VMEM((depth, D, block_k), jnp.float32),
            pltpu.VMEM((depth, D, block_k), jnp.float32),
            pltpu.VMEM((depth, D, block_k), jnp.float32),
            pltpu.VMEM((depth, block_k, 2 * D), jnp.float32),
            pltpu.SemaphoreType.DMA((depth, 4)),
            pltpu.SemaphoreType.DMA((depth,)),
        ],
    )(mt, qt)


def _sc_gather_packed(sum2, idx, half):
    B = idx.shape[0]
    L2 = sum2.shape[1]
    b_per_w = B // _NW
    assert B % (8 * _NW) == 0

    mesh = plsc.VectorSubcoreMesh(core_axis_name="c", subcore_axis_name="s")

    @functools.partial(
        pl.kernel,
        out_type=jax.ShapeDtypeStruct((B, L2), jnp.float32),
        mesh=mesh,
        scratch_types=[
            pltpu.VMEM((b_per_w,), jnp.int32),
            pltpu.VMEM((b_per_w,), jnp.int32),
            pltpu.VMEM((b_per_w, L2), jnp.float32),
            pltpu.SemaphoreType.DMA,
        ],
    )
    def gather_kernel(sum2_hbm, idx_hbm, g2_hbm, idx_v, idx2_v, g_v, sem):
        wid = lax.axis_index("s") * _NUM_CORES + lax.axis_index("c")
        base = wid * b_per_w
        pltpu.sync_copy(idx_hbm.at[pl.ds(base, b_per_w)], idx_v)
        for i in range(b_per_w // 16):
            v = idx_v[pl.ds(i * 16, 16)]
            idx2_v[pl.ds(i * 16, 16)] = jnp.where(v >= half, v - half, v)
        pltpu.async_copy(sum2_hbm.at[idx2_v], g_v, sem).wait()
        pltpu.sync_copy(g_v, g2_hbm.at[pl.ds(base, b_per_w)])

    return gather_kernel(sum2, idx)


def _tc_subtract_t(pt, g2, idx, tail_mq, half, v_edge, n_chunks, depth):
    T, D, B = pt.shape
    n_tail = tail_mq.shape[1]
    ch = T // n_chunks

    def body(p_hbm, g2_ref, idx_ref, tail_ref, o_hbm, pbuf, obuf,
             in_sems, out_sems):
        par = idx_ref[...][:, None]
        sel = jnp.where(par >= half, g2_ref[:, D:2 * D], g2_ref[:, 0:D])
        c = jnp.transpose(sel * 0.5, (1, 0))
        oh = (lax.broadcasted_iota(jnp.int32, (n_tail, B), 0)
              == (idx_ref[...] - v_edge)[None, :]).astype(jnp.float32)
        cfix = lax.dot_general(tail_ref[...] * 0.5, oh,
                               (((1,), (0,)), ((), ())),
                               preferred_element_type=jnp.float32)
        is_tail = (idx_ref[...] >= v_edge)[None, :]
        c = jnp.where(is_tail, cfix, c)

        in_cps = [
            pltpu.make_async_copy(
                p_hbm.at[pl.ds(j * ch, ch)], pbuf.at[j % depth],
                in_sems.at[j % depth])
            for j in range(n_chunks)
        ]
        out_cps = [
            pltpu.make_async_copy(
                obuf.at[j % depth], o_hbm.at[pl.ds(j * ch, ch)],
                out_sems.at[j % depth])
            for j in range(n_chunks)
        ]
        for j in range(depth):
            in_cps[j].start()
        for j in range(n_chunks):
            in_cps[j].wait()
            if j >= depth:
                out_cps[j - depth].wait()
            obuf[j % depth] = pbuf[j % depth] - c[None, :, :]
            out_cps[j].start()
            if j + depth < n_chunks:
                in_cps[j + depth].start()
        for j in range(n_chunks - depth, n_chunks):
            out_cps[j].wait()

    return pl.pallas_call(
        body,
        in_specs=[
            pl.BlockSpec(memory_space=pl.ANY),
            pl.BlockSpec((B, 2 * D), lambda: (0, 0)),
            pl.BlockSpec((B,), lambda: (0,)),
            pl.BlockSpec((D, n_tail), lambda: (0, 0)),
        ],
        out_specs=pl.BlockSpec(memory_space=pl.ANY),
        out_shape=jax.ShapeDtypeStruct((T, D, B), jnp.float32),
        scratch_shapes=[
            pltpu.VMEM((depth, ch, D, B), jnp.float32),
            pltpu.VMEM((depth, ch, D, B), jnp.float32),
            pltpu.SemaphoreType.DMA((depth,)),
            pltpu.SemaphoreType.DMA((depth,)),
        ],
    )(pt, g2, idx, tail_mq)


def kernel(prosody, spkr_id, means, question):
    idx = spkr_id.astype(jnp.int32)
    mt = jnp.transpose(means, (1, 0))
    qt = jnp.transpose(question, (1, 0))
    V = mt.shape[1]
    half = 51200
    v_edge = (V // 128) * 128
    sum2 = _tc_pack_sum(mt, qt, half=half, block_k=3200)
    g2 = _sc_gather_packed(sum2, idx, half=half)
    pt = jnp.transpose(prosody, (1, 2, 0))
    tail_mq = (lax.slice(mt, (0, v_edge), mt.shape)
               + lax.slice(qt, (0, v_edge), qt.shape))
    out_t = _tc_subtract_t(pt, g2, idx, tail_mq, half=half, v_edge=v_edge,
                           n_chunks=50, depth=10)
    return jnp.transpose(out_t, (2, 0, 1))


# speedup vs baseline: 1.1093x; 1.0009x over previous
import functools

import jax
import jax.numpy as jnp
from jax import lax
from jax.experimental import pallas as pl
from jax.experimental.pallas import tpu as pltpu
from jax.experimental.pallas import tpu_sc as plsc

_NUM_CORES = 2
_NUM_SUBCORES = 16
_NW = _NUM_CORES * _NUM_SUBCORES


def _tc_pack_sum(mt, qt, half, block_k):
    D, V = mt.shape
    grid = half // block_k
    depth = 4
    v_edge = (V // 128) * 128

    def body(mt_hbm, qt_hbm, o_hbm, mabuf, qabuf, mbbuf, qbbuf, obuf,
             in_sems, out_sems):
        def in_slice(tab, col0, buf, slot, sem):
            width = min(block_k, v_edge - col0) if col0 < v_edge else 0
            if width <= 0:
                return None
            return pltpu.make_async_copy(
                tab.at[:, pl.ds(col0, width)],
                buf.at[slot, :, pl.ds(0, width)], sem)

        in_cps = []
        for j in range(grid):
            slot = j % depth
            cps = [
                in_slice(mt_hbm, j * block_k, mabuf, slot, in_sems.at[slot, 0]),
                in_slice(qt_hbm, j * block_k, qabuf, slot, in_sems.at[slot, 1]),
                in_slice(mt_hbm, half + j * block_k, mbbuf, slot, in_sems.at[slot, 2]),
                in_slice(qt_hbm, half + j * block_k, qbbuf, slot, in_sems.at[slot, 3]),
            ]
            in_cps.append([c for c in cps if c is not None])
        out_cps = [
            pltpu.make_async_copy(
                obuf.at[j % depth], o_hbm.at[pl.ds(j * block_k, block_k)],
                out_sems.at[j % depth])
            for j in range(grid)
        ]
        for j in range(depth):
            for c in in_cps[j]:
                c.start()
        for j in range(grid):
            slot = j % depth
            for c in in_cps[j]:
                c.wait()
            if j >= depth:
                out_cps[j - depth].wait()
            ta = jnp.transpose(mabuf[slot] + qabuf[slot], (1, 0))
            tb = jnp.transpose(mbbuf[slot] + qbbuf[slot], (1, 0))
            obuf[slot] = jnp.concatenate([ta, tb], axis=1)
            out_cps[j].start()
            if j + depth < grid:
                for c in in_cps[j + depth]:
                    c.start()
        for j in range(grid - depth, grid):
            out_cps[j].wait()

    return pl.pallas_call(
        body,
        in_specs=[
            pl.BlockSpec(memory_space=pl.ANY),
            pl.BlockSpec(memory_space=pl.ANY),
        ],
        out_specs=pl.BlockSpec(memory_space=pl.ANY),
        out_shape=jax.ShapeDtypeStruct((half, 2 * D), jnp.float32),
        scratch_shapes=[
            pltpu.VMEM((depth, D, block_k), jnp.float32),
            pltpu.VMEM((depth, D, block_k), jnp.float32),
            pltpu.VMEM((depth, D, block_k), jnp.float32),
            pltpu.VMEM((depth, D, block_k), jnp.float32),
            pltpu.VMEM((depth, block_k, 2 * D), jnp.float32),
            pltpu.SemaphoreType.DMA((depth, 4)),
            pltpu.SemaphoreType.DMA((depth,)),
        ],
    )(mt, qt)


def _sc_gather_packed(sum2, idx, half):
    B = idx.shape[0]
    L2 = sum2.shape[1]
    b_per_w = B // _NW
    assert B % (8 * _NW) == 0

    mesh = plsc.VectorSubcoreMesh(core_axis_name="c", subcore_axis_name="s")

    @functools.partial(
        pl.kernel,
        out_type=jax.ShapeDtypeStruct((B, L2), jnp.float32),
        mesh=mesh,
        scratch_types=[
            pltpu.VMEM((b_per_w,), jnp.int32),
            pltpu.VMEM((b_per_w,), jnp.int32),
            pltpu.VMEM((b_per_w, L2), jnp.float32),
            pltpu.SemaphoreType.DMA,
        ],
    )
    def gather_kernel(sum2_hbm, idx_hbm, g2_hbm, idx_v, idx2_v, g_v, sem):
        wid = lax.axis_index("s") * _NUM_CORES + lax.axis_index("c")
        base = wid * b_per_w
        pltpu.sync_copy(idx_hbm.at[pl.ds(base, b_per_w)], idx_v)
        for i in range(b_per_w // 16):
            v = idx_v[pl.ds(i * 16, 16)]
            idx2_v[pl.ds(i * 16, 16)] = jnp.where(v >= half, v - half, v)
        pltpu.async_copy(sum2_hbm.at[idx2_v], g_v, sem).wait()
        pltpu.sync_copy(g_v, g2_hbm.at[pl.ds(base, b_per_w)])

    return gather_kernel(sum2, idx)


def _tc_subtract_t(pt, g2, idx, tail_mq, half, v_edge, n_chunks, depth):
    T, D, B = pt.shape
    n_tail = tail_mq.shape[1]
    ch = T // n_chunks

    def body(p_hbm, g2_ref, idx_ref, tail_ref, o_hbm, pbuf, obuf,
             in_sems, out_sems):
        par = idx_ref[...][:, None]
        sel = jnp.where(par >= half, g2_ref[:, D:2 * D], g2_ref[:, 0:D])
        c = jnp.transpose(sel * 0.5, (1, 0))
        oh = (lax.broadcasted_iota(jnp.int32, (n_tail, B), 0)
              == (idx_ref[...] - v_edge)[None, :]).astype(jnp.float32)
        cfix = lax.dot_general(tail_ref[...] * 0.5, oh,
                               (((1,), (0,)), ((), ())),
                               preferred_element_type=jnp.float32)
        is_tail = (idx_ref[...] >= v_edge)[None, :]
        c = jnp.where(is_tail, cfix, c)

        in_cps = [
            pltpu.make_async_copy(
                p_hbm.at[pl.ds(j * ch, ch)], pbuf.at[j % depth],
                in_sems.at[j % depth])
            for j in range(n_chunks)
        ]
        out_cps = [
            pltpu.make_async_copy(
                obuf.at[j % depth], o_hbm.at[pl.ds(j * ch, ch)],
                out_sems.at[j % depth])
            for j in range(n_chunks)
        ]
        for j in range(depth):
            in_cps[j].start()
        for j in range(n_chunks):
            in_cps[j].wait()
            if j >= depth:
                out_cps[j - depth].wait()
            obuf[j % depth] = pbuf[j % depth] - c[None, :, :]
            out_cps[j].start()
            if j + depth < n_chunks:
                in_cps[j + depth].start()
        for j in range(n_chunks - depth, n_chunks):
            out_cps[j].wait()

    return pl.pallas_call(
        body,
        in_specs=[
            pl.BlockSpec(memory_space=pl.ANY),
            pl.BlockSpec((B, 2 * D), lambda: (0, 0)),
            pl.BlockSpec((B,), lambda: (0,)),
            pl.BlockSpec((D, n_tail), lambda: (0, 0)),
        ],
        out_specs=pl.BlockSpec(memory_space=pl.ANY),
        out_shape=jax.ShapeDtypeStruct((T, D, B), jnp.float32),
        scratch_shapes=[
            pltpu.VMEM((depth, ch, D, B), jnp.float32),
            pltpu.VMEM((depth, ch, D, B), jnp.float32),
            pltpu.SemaphoreType.DMA((depth,)),
            pltpu.SemaphoreType.DMA((depth,)),
        ],
    )(pt, g2, idx, tail_mq)


def kernel(prosody, spkr_id, means, question):
    idx = spkr_id.astype(jnp.int32)
    mt = jnp.transpose(means, (1, 0))
    qt = jnp.transpose(question, (1, 0))
    V = mt.shape[1]
    half = 51200
    v_edge = (V // 128) * 128
    sum2 = _tc_pack_sum(mt, qt, half=half, block_k=3200)
    g2 = _sc_gather_packed(sum2, idx, half=half)
    pt = jnp.transpose(prosody, (1, 2, 0))
    tail_mq = (lax.slice(mt, (0, v_edge), mt.shape)
               + lax.slice(qt, (0, v_edge), qt.shape))
    out_t = _tc_subtract_t(pt, g2, idx, tail_mq, half=half, v_edge=v_edge,
                           n_chunks=50, depth=10)
    return jnp.transpose(out_t, (2, 0, 1))
